# baseline (speedup 1.0000x reference)
import jax
import jax.numpy as jnp
from jax.experimental import pallas as pl

_EPS = 1e-12
_GRID = 32


def _norm_kernel(z_ref, c_ref, oz_ref, oc_ref):
    z = z_ref[...]
    n = jnp.sqrt(jnp.sum(z * z, axis=1, keepdims=True))
    oz_ref[...] = z / jnp.maximum(n, _EPS)
    c = c_ref[...]
    m = jnp.sqrt(jnp.sum(c * c, axis=1, keepdims=True))
    oc_ref[...] = c / jnp.maximum(m, _EPS)


def kernel(z, centroids):
    bz = z.shape[0] // _GRID
    bc = centroids.shape[0] // _GRID
    d = z.shape[1]
    return pl.pallas_call(
        _norm_kernel,
        grid=(_GRID,),
        in_specs=[
            pl.BlockSpec((bz, d), lambda i: (i, 0)),
            pl.BlockSpec((bc, d), lambda i: (i, 0)),
        ],
        out_specs=[
            pl.BlockSpec((bz, d), lambda i: (i, 0)),
            pl.BlockSpec((bc, d), lambda i: (i, 0)),
        ],
        out_shape=[
            jax.ShapeDtypeStruct(z.shape, z.dtype),
            jax.ShapeDtypeStruct(centroids.shape, centroids.dtype),
        ],
    )(z, centroids)


# speedup vs baseline: 1.6666x; 1.6666x over previous
import jax
import jax.numpy as jnp
from jax.experimental import pallas as pl

_EPS = 1e-12
_GRID = 8


def _norm_kernel(z_ref, c_ref, oz_ref, oc_ref):
    z = z_ref[...]
    n = jnp.sqrt(jnp.sum(z * z, axis=1, keepdims=True))
    oz_ref[...] = z / jnp.maximum(n, _EPS)
    c = c_ref[...]
    m = jnp.sqrt(jnp.sum(c * c, axis=1, keepdims=True))
    oc_ref[...] = c / jnp.maximum(m, _EPS)


def kernel(z, centroids):
    bz = z.shape[0] // _GRID
    bc = centroids.shape[0] // _GRID
    d = z.shape[1]
    return pl.pallas_call(
        _norm_kernel,
        grid=(_GRID,),
        in_specs=[
            pl.BlockSpec((bz, d), lambda i: (i, 0)),
            pl.BlockSpec((bc, d), lambda i: (i, 0)),
        ],
        out_specs=[
            pl.BlockSpec((bz, d), lambda i: (i, 0)),
            pl.BlockSpec((bc, d), lambda i: (i, 0)),
        ],
        out_shape=[
            jax.ShapeDtypeStruct(z.shape, z.dtype),
            jax.ShapeDtypeStruct(centroids.shape, centroids.dtype),
        ],
    )(z, centroids)


# speedup vs baseline: 1.7334x; 1.0401x over previous
import jax
import jax.numpy as jnp
from jax.experimental import pallas as pl

_EPS = 1e-12
_GRID = 4


def _norm_kernel(z_ref, c_ref, oz_ref, oc_ref):
    z = z_ref[...]
    n = jnp.sqrt(jnp.sum(z * z, axis=1, keepdims=True))
    oz_ref[...] = z / jnp.maximum(n, _EPS)
    c = c_ref[...]
    m = jnp.sqrt(jnp.sum(c * c, axis=1, keepdims=True))
    oc_ref[...] = c / jnp.maximum(m, _EPS)


def kernel(z, centroids):
    bz = z.shape[0] // _GRID
    bc = centroids.shape[0] // _GRID
    d = z.shape[1]
    return pl.pallas_call(
        _norm_kernel,
        grid=(_GRID,),
        in_specs=[
            pl.BlockSpec((bz, d), lambda i: (i, 0)),
            pl.BlockSpec((bc, d), lambda i: (i, 0)),
        ],
        out_specs=[
            pl.BlockSpec((bz, d), lambda i: (i, 0)),
            pl.BlockSpec((bc, d), lambda i: (i, 0)),
        ],
        out_shape=[
            jax.ShapeDtypeStruct(z.shape, z.dtype),
            jax.ShapeDtypeStruct(centroids.shape, centroids.dtype),
        ],
    )(z, centroids)


# speedup vs baseline: 1.9663x; 1.1344x over previous
import jax
import jax.numpy as jnp
from jax.experimental import pallas as pl

_EPS = 1e-12
_GRID = 2


def _norm_kernel(z_ref, c_ref, oz_ref, oc_ref):
    z = z_ref[...]
    n = jnp.sqrt(jnp.sum(z * z, axis=1, keepdims=True))
    oz_ref[...] = z / jnp.maximum(n, _EPS)
    c = c_ref[...]
    m = jnp.sqrt(jnp.sum(c * c, axis=1, keepdims=True))
    oc_ref[...] = c / jnp.maximum(m, _EPS)


def kernel(z, centroids):
    bz = z.shape[0] // _GRID
    bc = centroids.shape[0] // _GRID
    d = z.shape[1]
    return pl.pallas_call(
        _norm_kernel,
        grid=(_GRID,),
        in_specs=[
            pl.BlockSpec((bz, d), lambda i: (i, 0)),
            pl.BlockSpec((bc, d), lambda i: (i, 0)),
        ],
        out_specs=[
            pl.BlockSpec((bz, d), lambda i: (i, 0)),
            pl.BlockSpec((bc, d), lambda i: (i, 0)),
        ],
        out_shape=[
            jax.ShapeDtypeStruct(z.shape, z.dtype),
            jax.ShapeDtypeStruct(centroids.shape, centroids.dtype),
        ],
    )(z, centroids)


# speedup vs baseline: 1.9691x; 1.0014x over previous
import jax
import jax.numpy as jnp
from jax.experimental import pallas as pl

_EPS = 1e-12
_GRID = 2


def _norm_kernel(z_ref, c_ref, oz_ref, oc_ref):
    z = z_ref[...]
    n = jnp.sqrt(jnp.sum(z * z, axis=1, keepdims=True))
    oz_ref[...] = z * (1.0 / jnp.maximum(n, _EPS))
    c = c_ref[...]
    m = jnp.sqrt(jnp.sum(c * c, axis=1, keepdims=True))
    oc_ref[...] = c * (1.0 / jnp.maximum(m, _EPS))


def kernel(z, centroids):
    bz = z.shape[0] // _GRID
    bc = centroids.shape[0] // _GRID
    d = z.shape[1]
    return pl.pallas_call(
        _norm_kernel,
        grid=(_GRID,),
        in_specs=[
            pl.BlockSpec((bz, d), lambda i: (i, 0)),
            pl.BlockSpec((bc, d), lambda i: (i, 0)),
        ],
        out_specs=[
            pl.BlockSpec((bz, d), lambda i: (i, 0)),
            pl.BlockSpec((bc, d), lambda i: (i, 0)),
        ],
        out_shape=[
            jax.ShapeDtypeStruct(z.shape, z.dtype),
            jax.ShapeDtypeStruct(centroids.shape, centroids.dtype),
        ],
    )(z, centroids)
